# baseline (speedup 1.0000x reference)
import functools

import jax
import jax.numpy as jnp
from jax import lax
from jax.experimental import pallas as pl
from jax.experimental.pallas import tpu as pltpu
from jax.experimental.pallas import tpu_sc as plsc

NUM_RAW = 8100
NUM_POL = 2550
BATCH = 4096

NC = 2
NS = 16
L = 16

K_IDX = (NUM_RAW + L - 1) // L
K_OUT = (NUM_POL + L - 1) // L
SRC_PAD = K_OUT * L
NBUF = 64


def _src_body(idx_hbm, src_hbm, idx_v, src_v):
    wid = lax.axis_index("s") * NC + lax.axis_index("c")

    @pl.when(wid == 0)
    def _build():
        lane = lax.broadcasted_iota(jnp.int32, (L,), 0)
        pltpu.sync_copy(idx_hbm, idx_v.at[pl.ds(0, NUM_RAW)])
        src_v[pl.ds(SRC_PAD - L, L)] = jnp.zeros((L,), jnp.int32)

        def build_src(k, _):
            vec = idx_v[pl.ds(k * L, L)]
            j = k * L + lane
            m = (vec >= 0) & (j < NUM_RAW)
            addr = jnp.where(m, vec, 0)
            plsc.store_scatter(src_v, [addr], j, mask=m)
            return 0

        lax.fori_loop(0, K_IDX, build_src, 0)
        pltpu.sync_copy(src_v, src_hbm)


def _gather_body(src_smem, xt_hbm, out_hbm, sems):
    def copy_for(p):
        s = src_smem[p]
        return pltpu.make_async_copy(
            xt_hbm.at[pl.ds(s, 1)], out_hbm.at[pl.ds(p, 1)], sems.at[p % NBUF])

    def step(p, _):
        @pl.when(p >= NBUF)
        def _drain():
            copy_for(p - NBUF).wait()

        copy_for(p).start()
        return 0

    lax.fori_loop(0, NUM_POL, step, 0)

    def drain(p, _):
        copy_for(p).wait()
        return 0

    lax.fori_loop(NUM_POL - NBUF, NUM_POL, drain, 0)


@jax.jit
def kernel(policy_logits_8100, policy_index_array):
    idx32 = policy_index_array.astype(jnp.int32)
    xt = policy_logits_8100.T

    mesh = plsc.VectorSubcoreMesh(
        core_axis_name="c", subcore_axis_name="s", num_cores=NC, num_subcores=NS
    )
    src = pl.kernel(
        _src_body,
        out_type=jax.ShapeDtypeStruct((SRC_PAD,), jnp.int32),
        mesh=mesh,
        scratch_types=[
            pltpu.VMEM((K_IDX * L,), jnp.int32),
            pltpu.VMEM((SRC_PAD,), jnp.int32),
        ],
        compiler_params=pltpu.CompilerParams(needs_layout_passes=False),
    )(idx32)

    grid_spec = pltpu.PrefetchScalarGridSpec(
        num_scalar_prefetch=1,
        grid=(1,),
        in_specs=[pl.BlockSpec(memory_space=pl.ANY)],
        out_specs=pl.BlockSpec(memory_space=pl.ANY),
        scratch_shapes=[pltpu.SemaphoreType.DMA((NBUF,))],
    )
    xt3 = xt.reshape(NUM_RAW, 8, BATCH // 8)
    out3 = pl.pallas_call(
        _gather_body,
        grid_spec=grid_spec,
        out_shape=jax.ShapeDtypeStruct((NUM_POL, 8, BATCH // 8), jnp.float32),
    )(src, xt3)

    return out3.reshape(NUM_POL, BATCH).T


# speedup vs baseline: 12.4642x; 12.4642x over previous
import jax
import jax.numpy as jnp
from jax import lax
from jax.experimental import pallas as pl
from jax.experimental.pallas import tpu as pltpu
from jax.experimental.pallas import tpu_sc as plsc

NUM_RAW = 8100
NUM_POL = 2550
BATCH = 4096

NC = 2
NS = 16
L = 16

K_IDX = (NUM_RAW + L - 1) // L
K_OUT = (NUM_POL + L - 1) // L
SRC_PAD = K_OUT * L
BCHUNK = 512


def _src_body(idx_hbm, src_hbm, idx_v, src_v):
    wid = lax.axis_index("s") * NC + lax.axis_index("c")

    @pl.when(wid == 0)
    def _build():
        lane = lax.broadcasted_iota(jnp.int32, (L,), 0)
        pltpu.sync_copy(idx_hbm, idx_v.at[pl.ds(0, NUM_RAW)])
        src_v[pl.ds(SRC_PAD - L, L)] = jnp.zeros((L,), jnp.int32)

        def build_src(k, _):
            vec = idx_v[pl.ds(k * L, L)]
            j = k * L + lane
            m = (vec >= 0) & (j < NUM_RAW)
            addr = jnp.where(m, vec, 0)
            plsc.store_scatter(src_v, [addr], j, mask=m)
            return 0

        lax.fori_loop(0, K_IDX, build_src, 0)
        pltpu.sync_copy(src_v, src_hbm)


def _gather_body(src_smem, xt_ref, out_ref):
    def step(p, _):
        s = src_smem[p]
        out_ref[pl.ds(p, 1), :] = xt_ref[pl.ds(s, 1), :]
        return 0

    lax.fori_loop(0, NUM_POL, step, 0, unroll=4)


@jax.jit
def kernel(policy_logits_8100, policy_index_array):
    idx32 = policy_index_array.astype(jnp.int32)
    xt = policy_logits_8100.T

    mesh = plsc.VectorSubcoreMesh(
        core_axis_name="c", subcore_axis_name="s", num_cores=NC, num_subcores=NS
    )
    src = pl.kernel(
        _src_body,
        out_type=jax.ShapeDtypeStruct((SRC_PAD,), jnp.int32),
        mesh=mesh,
        scratch_types=[
            pltpu.VMEM((K_IDX * L,), jnp.int32),
            pltpu.VMEM((SRC_PAD,), jnp.int32),
        ],
        compiler_params=pltpu.CompilerParams(needs_layout_passes=False),
    )(idx32)

    grid_spec = pltpu.PrefetchScalarGridSpec(
        num_scalar_prefetch=1,
        grid=(BATCH // BCHUNK,),
        in_specs=[
            pl.BlockSpec((NUM_RAW, BCHUNK), lambda i, *_: (0, i)),
        ],
        out_specs=pl.BlockSpec((NUM_POL, BCHUNK), lambda i, *_: (0, i)),
    )
    out_t = pl.pallas_call(
        _gather_body,
        grid_spec=grid_spec,
        out_shape=jax.ShapeDtypeStruct((NUM_POL, BATCH), jnp.float32),
    )(src, xt)

    return out_t.T


# speedup vs baseline: 14.2207x; 1.1409x over previous
import jax
import jax.numpy as jnp
from jax import lax
from jax.experimental import pallas as pl
from jax.experimental.pallas import tpu as pltpu
from jax.experimental.pallas import tpu_sc as plsc

NUM_RAW = 8100
NUM_POL = 2550
BATCH = 4096

NC = 2
NS = 16
L = 16

K_IDX = (NUM_RAW + L - 1) // L
K_OUT = (NUM_POL + L - 1) // L
SRC_PAD = K_OUT * L
BCHUNK = 512


def _src_body(idx_hbm, src_hbm, idx_v, src_v):
    wid = lax.axis_index("s") * NC + lax.axis_index("c")

    @pl.when(wid == 0)
    def _build():
        lane = lax.broadcasted_iota(jnp.int32, (L,), 0)
        pltpu.sync_copy(idx_hbm, idx_v.at[pl.ds(0, NUM_RAW)])
        src_v[pl.ds(SRC_PAD - L, L)] = jnp.zeros((L,), jnp.int32)

        def build_src(k, _):
            vec = idx_v[pl.ds(k * L, L)]
            j = k * L + lane
            m = (vec >= 0) & (j < NUM_RAW)
            addr = jnp.where(m, vec, 0)
            plsc.store_scatter(src_v, [addr], j, mask=m)
            return 0

        lax.fori_loop(0, K_IDX, build_src, 0)
        pltpu.sync_copy(src_v, src_hbm)


def _gather_body(src_smem, xt_ref, out_ref):
    def step(p, _):
        s = src_smem[p]
        out_ref[pl.ds(p, 1), :] = xt_ref[pl.ds(s, 1), :]
        return 0

    lax.fori_loop(0, NUM_POL, step, 0, unroll=8)


@jax.jit
def kernel(policy_logits_8100, policy_index_array):
    idx32 = policy_index_array.astype(jnp.int32)
    xt = policy_logits_8100.T

    mesh = plsc.VectorSubcoreMesh(
        core_axis_name="c", subcore_axis_name="s", num_cores=NC, num_subcores=NS
    )
    src = pl.kernel(
        _src_body,
        out_type=jax.ShapeDtypeStruct((SRC_PAD,), jnp.int32),
        mesh=mesh,
        scratch_types=[
            pltpu.VMEM((K_IDX * L,), jnp.int32),
            pltpu.VMEM((SRC_PAD,), jnp.int32),
        ],
        compiler_params=pltpu.CompilerParams(needs_layout_passes=False),
    )(idx32)

    grid_spec = pltpu.PrefetchScalarGridSpec(
        num_scalar_prefetch=1,
        grid=(BATCH // BCHUNK,),
        in_specs=[
            pl.BlockSpec((NUM_RAW, BCHUNK), lambda i, *_: (0, i)),
        ],
        out_specs=pl.BlockSpec((NUM_POL, BCHUNK), lambda i, *_: (0, i)),
    )
    out_t = pl.pallas_call(
        _gather_body,
        grid_spec=grid_spec,
        out_shape=jax.ShapeDtypeStruct((NUM_POL, BATCH), jnp.float32),
        compiler_params=pltpu.CompilerParams(
            vmem_limit_bytes=115 * 1024 * 1024),
    )(src, xt)

    return out_t.T


# speedup vs baseline: 14.9501x; 1.0513x over previous
import jax
import jax.numpy as jnp
from jax import lax
from jax.experimental import pallas as pl
from jax.experimental.pallas import tpu as pltpu
from jax.experimental.pallas import tpu_sc as plsc

NUM_RAW = 8100
NUM_POL = 2550
BATCH = 4096

NC = 2
NS = 16
L = 16

K_IDX = (NUM_RAW + L - 1) // L
K_OUT = (NUM_POL + L - 1) // L
SRC_PAD = K_OUT * L
BCHUNK = 512


def _src_body(idx_hbm, src_hbm, idx_v, src_v):
    wid = lax.axis_index("s") * NC + lax.axis_index("c")

    @pl.when(wid == 0)
    def _build():
        lane = lax.broadcasted_iota(jnp.int32, (L,), 0)
        pltpu.sync_copy(idx_hbm, idx_v.at[pl.ds(0, NUM_RAW)])
        src_v[pl.ds(SRC_PAD - L, L)] = jnp.zeros((L,), jnp.int32)

        def build_src(k, _):
            vec = idx_v[pl.ds(k * L, L)]
            j = k * L + lane
            m = (vec >= 0) & (j < NUM_RAW)
            addr = jnp.where(m, vec, 0)
            plsc.store_scatter(src_v, [addr], j, mask=m)
            return 0

        lax.fori_loop(0, K_IDX, build_src, 0)
        pltpu.sync_copy(src_v, src_hbm)


def _gather_body(src_smem, xt_ref, out_ref):
    def step(p, _):
        s = src_smem[p]
        out_ref[pl.ds(p, 1), :] = xt_ref[pl.ds(s, 1), :]
        return 0

    lax.fori_loop(0, NUM_POL, step, 0, unroll=16)


@jax.jit
def kernel(policy_logits_8100, policy_index_array):
    idx32 = policy_index_array.astype(jnp.int32)
    xt = policy_logits_8100.T

    mesh = plsc.VectorSubcoreMesh(
        core_axis_name="c", subcore_axis_name="s", num_cores=NC, num_subcores=NS
    )
    src = pl.kernel(
        _src_body,
        out_type=jax.ShapeDtypeStruct((SRC_PAD,), jnp.int32),
        mesh=mesh,
        scratch_types=[
            pltpu.VMEM((K_IDX * L,), jnp.int32),
            pltpu.VMEM((SRC_PAD,), jnp.int32),
        ],
        compiler_params=pltpu.CompilerParams(needs_layout_passes=False),
    )(idx32)

    grid_spec = pltpu.PrefetchScalarGridSpec(
        num_scalar_prefetch=1,
        grid=(BATCH // BCHUNK,),
        in_specs=[
            pl.BlockSpec((NUM_RAW, BCHUNK), lambda i, *_: (0, i)),
        ],
        out_specs=pl.BlockSpec((NUM_POL, BCHUNK), lambda i, *_: (0, i)),
    )
    out_t = pl.pallas_call(
        _gather_body,
        grid_spec=grid_spec,
        out_shape=jax.ShapeDtypeStruct((NUM_POL, BATCH), jnp.float32),
        compiler_params=pltpu.CompilerParams(
            vmem_limit_bytes=115 * 1024 * 1024),
    )(src, xt)

    return out_t.T


# speedup vs baseline: 15.3338x; 1.0257x over previous
import jax
import jax.numpy as jnp
from jax import lax
from jax.experimental import pallas as pl
from jax.experimental.pallas import tpu as pltpu
from jax.experimental.pallas import tpu_sc as plsc

NUM_RAW = 8100
NUM_POL = 2550
BATCH = 4096

NC = 2
NS = 16
L = 16

K_IDX = (NUM_RAW + L - 1) // L
K_OUT = (NUM_POL + L - 1) // L
SRC_PAD = K_OUT * L
BCHUNK = 512


def _src_body(idx_hbm, src_hbm, idx_v, src_v):
    wid = lax.axis_index("s") * NC + lax.axis_index("c")

    @pl.when(wid == 0)
    def _build():
        lane = lax.broadcasted_iota(jnp.int32, (L,), 0)
        pltpu.sync_copy(idx_hbm, idx_v.at[pl.ds(0, NUM_RAW)])
        src_v[pl.ds(SRC_PAD - L, L)] = jnp.zeros((L,), jnp.int32)

        def build_src(k, _):
            vec = idx_v[pl.ds(k * L, L)]
            j = k * L + lane
            m = (vec >= 0) & (j < NUM_RAW)
            addr = jnp.where(m, vec, 0)
            plsc.store_scatter(src_v, [addr], j, mask=m)
            return 0

        lax.fori_loop(0, K_IDX, build_src, 0)
        pltpu.sync_copy(src_v, src_hbm)


def _gather_body(src_smem, xt_ref, out_ref):
    def step(p, _):
        s = src_smem[p]
        out_ref[pl.ds(p, 1), :] = xt_ref[pl.ds(s, 1), :]
        return 0

    lax.fori_loop(0, NUM_POL, step, 0, unroll=32)


@jax.jit
def kernel(policy_logits_8100, policy_index_array):
    idx32 = policy_index_array.astype(jnp.int32)
    xt = policy_logits_8100.T

    mesh = plsc.VectorSubcoreMesh(
        core_axis_name="c", subcore_axis_name="s", num_cores=NC, num_subcores=NS
    )
    src = pl.kernel(
        _src_body,
        out_type=jax.ShapeDtypeStruct((SRC_PAD,), jnp.int32),
        mesh=mesh,
        scratch_types=[
            pltpu.VMEM((K_IDX * L,), jnp.int32),
            pltpu.VMEM((SRC_PAD,), jnp.int32),
        ],
        compiler_params=pltpu.CompilerParams(needs_layout_passes=False),
    )(idx32)

    grid_spec = pltpu.PrefetchScalarGridSpec(
        num_scalar_prefetch=1,
        grid=(BATCH // BCHUNK,),
        in_specs=[
            pl.BlockSpec((NUM_RAW, BCHUNK), lambda i, *_: (0, i)),
        ],
        out_specs=pl.BlockSpec((NUM_POL, BCHUNK), lambda i, *_: (0, i)),
    )
    out_t = pl.pallas_call(
        _gather_body,
        grid_spec=grid_spec,
        out_shape=jax.ShapeDtypeStruct((NUM_POL, BATCH), jnp.float32),
        compiler_params=pltpu.CompilerParams(
            vmem_limit_bytes=115 * 1024 * 1024),
    )(src, xt)

    return out_t.T


# speedup vs baseline: 15.5943x; 1.0170x over previous
import jax
import jax.numpy as jnp
from jax import lax
from jax.experimental import pallas as pl
from jax.experimental.pallas import tpu as pltpu
from jax.experimental.pallas import tpu_sc as plsc

NUM_RAW = 8100
NUM_POL = 2550
BATCH = 4096

NC = 2
NS = 16
L = 16

K_IDX = (NUM_RAW + L - 1) // L
K_OUT = (NUM_POL + L - 1) // L
SRC_PAD = K_OUT * L
BCHUNK = 512


def _src_body(idx_hbm, src_hbm, idx_v, src_v):
    wid = lax.axis_index("s") * NC + lax.axis_index("c")

    @pl.when(wid == 0)
    def _build():
        lane = lax.broadcasted_iota(jnp.int32, (L,), 0)
        pltpu.sync_copy(idx_hbm, idx_v.at[pl.ds(0, NUM_RAW)])
        src_v[pl.ds(SRC_PAD - L, L)] = jnp.zeros((L,), jnp.int32)

        def build_src(k, _):
            vec = idx_v[pl.ds(k * L, L)]
            j = k * L + lane
            m = (vec >= 0) & (j < NUM_RAW)
            addr = jnp.where(m, vec, 0)
            plsc.store_scatter(src_v, [addr], j, mask=m)
            return 0

        lax.fori_loop(0, K_IDX, build_src, 0)
        pltpu.sync_copy(src_v, src_hbm)


def _gather_body(src_smem, xt_ref, out_ref):
    def step(p, _):
        s = src_smem[p]
        out_ref[pl.ds(p, 1), :] = xt_ref[pl.ds(s, 1), :]
        return 0

    lax.fori_loop(0, NUM_POL, step, 0, unroll=64)


@jax.jit
def kernel(policy_logits_8100, policy_index_array):
    idx32 = policy_index_array.astype(jnp.int32)
    xt = policy_logits_8100.T

    mesh = plsc.VectorSubcoreMesh(
        core_axis_name="c", subcore_axis_name="s", num_cores=NC, num_subcores=NS
    )
    src = pl.kernel(
        _src_body,
        out_type=jax.ShapeDtypeStruct((SRC_PAD,), jnp.int32),
        mesh=mesh,
        scratch_types=[
            pltpu.VMEM((K_IDX * L,), jnp.int32),
            pltpu.VMEM((SRC_PAD,), jnp.int32),
        ],
        compiler_params=pltpu.CompilerParams(needs_layout_passes=False),
    )(idx32)

    grid_spec = pltpu.PrefetchScalarGridSpec(
        num_scalar_prefetch=1,
        grid=(BATCH // BCHUNK,),
        in_specs=[
            pl.BlockSpec((NUM_RAW, BCHUNK), lambda i, *_: (0, i)),
        ],
        out_specs=pl.BlockSpec((NUM_POL, BCHUNK), lambda i, *_: (0, i)),
    )
    out_t = pl.pallas_call(
        _gather_body,
        grid_spec=grid_spec,
        out_shape=jax.ShapeDtypeStruct((NUM_POL, BATCH), jnp.float32),
        compiler_params=pltpu.CompilerParams(
            vmem_limit_bytes=115 * 1024 * 1024),
    )(src, xt)

    return out_t.T


# speedup vs baseline: 17.6868x; 1.1342x over previous
import jax
import jax.numpy as jnp
from jax import lax
from jax.experimental import pallas as pl
from jax.experimental.pallas import tpu as pltpu
from jax.experimental.pallas import tpu_sc as plsc

NUM_RAW = 8100
NUM_POL = 2550
BATCH = 4096

NC = 2
NS = 16
L = 16

K_IDX = (NUM_RAW + L - 1) // L
K_OUT = (NUM_POL + L - 1) // L
SRC_PAD = K_OUT * L
BCHUNK = 512


def _src_body(idx_hbm, src_hbm, idx_v, src_v):
    wid = lax.axis_index("s") * NC + lax.axis_index("c")

    @pl.when(wid == 0)
    def _build():
        lane = lax.broadcasted_iota(jnp.int32, (L,), 0)
        pltpu.sync_copy(idx_hbm, idx_v.at[pl.ds(0, NUM_RAW)])
        src_v[pl.ds(SRC_PAD - L, L)] = jnp.zeros((L,), jnp.int32)

        def build_src(k, _):
            vec = idx_v[pl.ds(k * L, L)]
            j = k * L + lane
            m = (vec >= 0) & (j < NUM_RAW)
            addr = jnp.where(m, vec, 0)
            plsc.store_scatter(src_v, [addr], j, mask=m)
            return 0

        lax.fori_loop(0, K_IDX, build_src, 0)
        pltpu.sync_copy(src_v, src_hbm)


def _gather_body(src_smem, xt_ref, out_ref):
    ngrp = NUM_POL // 8

    def step(g, _):
        base = g * 8
        rows = [xt_ref[pl.ds(src_smem[base + i], 1), :] for i in range(8)]
        out_ref[pl.ds(base, 8), :] = jnp.concatenate(rows, axis=0)
        return 0

    lax.fori_loop(0, ngrp, step, 0, unroll=8)

    for p in range(ngrp * 8, NUM_POL):
        out_ref[pl.ds(p, 1), :] = xt_ref[pl.ds(src_smem[p], 1), :]


@jax.jit
def kernel(policy_logits_8100, policy_index_array):
    idx32 = policy_index_array.astype(jnp.int32)
    xt = policy_logits_8100.T

    mesh = plsc.VectorSubcoreMesh(
        core_axis_name="c", subcore_axis_name="s", num_cores=NC, num_subcores=NS
    )
    src = pl.kernel(
        _src_body,
        out_type=jax.ShapeDtypeStruct((SRC_PAD,), jnp.int32),
        mesh=mesh,
        scratch_types=[
            pltpu.VMEM((K_IDX * L,), jnp.int32),
            pltpu.VMEM((SRC_PAD,), jnp.int32),
        ],
        compiler_params=pltpu.CompilerParams(needs_layout_passes=False),
    )(idx32)

    grid_spec = pltpu.PrefetchScalarGridSpec(
        num_scalar_prefetch=1,
        grid=(BATCH // BCHUNK,),
        in_specs=[
            pl.BlockSpec((NUM_RAW, BCHUNK), lambda i, *_: (0, i)),
        ],
        out_specs=pl.BlockSpec((NUM_POL, BCHUNK), lambda i, *_: (0, i)),
    )
    out_t = pl.pallas_call(
        _gather_body,
        grid_spec=grid_spec,
        out_shape=jax.ShapeDtypeStruct((NUM_POL, BATCH), jnp.float32),
        compiler_params=pltpu.CompilerParams(
            vmem_limit_bytes=115 * 1024 * 1024),
    )(src, xt)

    return out_t.T
